# Initial kernel scaffold; baseline (speedup 1.0000x reference)
import jax
import jax.numpy as jnp
from jax.experimental import pallas as pl


def kernel(x_s, x_t, W, gamma, beta):
    raise NotImplementedError("write your pallas kernel here")


# speedup vs baseline: 2.0081x; 2.0081x over previous
import functools

import jax
import jax.numpy as jnp
from jax import lax
from jax.experimental import pallas as pl
from jax.experimental.pallas import tpu as pltpu
from jax.experimental.pallas import tpu_sc as plsc

DIM = 128
L = 200
LP = 208
EPS = 1e-12

NCORES = 2
NSUB = 16
NW = NCORES * NSUB
NCH = DIM // 16


def _sc_pool(W, idx_flat, rows):
    rows_per_w = rows // NW
    mesh = plsc.VectorSubcoreMesh(core_axis_name="c", subcore_axis_name="s")

    @functools.partial(
        pl.kernel,
        out_type=jax.ShapeDtypeStruct((rows, DIM), jnp.float32),
        mesh=mesh,
        scratch_types=[
            pltpu.VMEM((LP,), jnp.int32),
            pltpu.VMEM((LP, DIM), jnp.float32),
            pltpu.VMEM((DIM,), jnp.float32),
            pltpu.SemaphoreType.DMA,
        ],
    )
    def pool_kernel(w_hbm, idx_hbm, out_hbm, idx_v, rows_v, orow_v, sem):
        wid = lax.axis_index("c") * NSUB + lax.axis_index("s")
        base = wid * rows_per_w

        @pl.loop(0, rows_per_w)
        def _(r):
            g = base + r
            off = pl.multiple_of(g * LP, 8)
            pltpu.sync_copy(idx_hbm.at[pl.ds(off, LP)], idx_v)
            pltpu.async_copy(w_hbm.at[idx_v], rows_v, sem).wait()

            def body(l, acc):
                return tuple(
                    acc[c] + rows_v[l, pl.ds(c * 16, 16)] for c in range(NCH)
                )

            acc = lax.fori_loop(
                0, LP, body,
                tuple(jnp.zeros((16,), jnp.float32) for _ in range(NCH)),
            )
            for c in range(NCH):
                orow_v[pl.ds(c * 16, 16)] = acc[c]
            pltpu.sync_copy(orow_v, out_hbm.at[g])

    return pool_kernel(W, idx_flat)


def _tc_norm(psum, idx, gamma, beta, rows):
    blk = 256

    def body(ps_ref, idx_ref, g_ref, b_ref, o_ref):
        s = ps_ref[...]
        cnt = jnp.sum((idx_ref[...] != 0).astype(jnp.float32), axis=1,
                      keepdims=True)
        p = s / cnt
        mu = jnp.mean(p, axis=1, keepdims=True)
        var = jnp.mean((p - mu) ** 2, axis=1, keepdims=True)
        o_ref[...] = (p - mu) * lax.rsqrt(var + EPS) * g_ref[...] + b_ref[...]

    return pl.pallas_call(
        body,
        grid=(rows // blk,),
        in_specs=[
            pl.BlockSpec((blk, DIM), lambda i: (i, 0)),
            pl.BlockSpec((blk, L), lambda i: (i, 0)),
            pl.BlockSpec((1, DIM), lambda i: (0, 0)),
            pl.BlockSpec((1, DIM), lambda i: (0, 0)),
        ],
        out_specs=pl.BlockSpec((blk, DIM), lambda i: (i, 0)),
        out_shape=jax.ShapeDtypeStruct((rows, DIM), jnp.float32),
    )(psum, idx, gamma.reshape(1, DIM), beta.reshape(1, DIM))


def kernel(x_s, x_t, W, gamma, beta):
    b = x_s.shape[0]
    rows = 2 * b
    idx = jnp.concatenate([x_s, x_t], axis=0)
    idx_flat = jnp.pad(idx, ((0, 0), (0, LP - L))).reshape(-1)
    psum = _sc_pool(W, idx_flat, rows)
    out = _tc_norm(psum, idx, gamma, beta, rows)
    return out[:b], out[b:]
